# baseline (speedup 1.0000x reference)
import functools

import jax
import jax.numpy as jnp
import numpy as np
from jax import lax
from jax.experimental import pallas as pl
from jax.experimental.pallas import tpu as pltpu
from jax.experimental.pallas import tpu_sc as plsc

F32 = jnp.float32

NC = 2
NS = 16
LANES = 16
NW = NC * NS
BATCH = 128


def _sc_mesh():
    return plsc.VectorSubcoreMesh(
        core_axis_name="c", subcore_axis_name="s",
        num_cores=NC, num_subcores=NS)


def _make_hist(n_pad, ub):
    npt = n_pad // NS
    ept = ub * BATCH

    fire = 16

    def body(edge_hbm, d0_hbm, d1_hbm, eflat, ridx, ones_v, zbuf, ssem,
             deg_sh):
        cid = lax.axis_index("c")
        sid = lax.axis_index("s")
        wid = cid * NS + sid
        for i in range(npt // LANES):
            zbuf[pl.ds(i * LANES, LANES)] = jnp.zeros((LANES,), F32)
        for i in range(BATCH // LANES):
            ones_v[pl.ds(i * LANES, LANES)] = jnp.ones((LANES,), F32)
        pltpu.sync_copy(zbuf, deg_sh.at[pl.ds(sid * npt, npt)])
        ebase = pl.multiple_of(wid * ept, BATCH)
        pltpu.sync_copy(edge_hbm.at[pl.ds(0, 1), pl.ds(ebase, ept)], eflat)

        def repack(j, c):
            for i in range(BATCH // LANES):
                ridx[j, pl.ds(i * LANES, LANES)] = (
                    eflat[0, pl.ds(j * BATCH + i * LANES, LANES)])
            return c

        lax.fori_loop(0, ub, repack, 0)
        plsc.subcore_barrier()

        def step(ci, c):
            for k in range(fire):
                pltpu.async_copy(
                    ones_v, deg_sh.at[ridx.at[ci * fire + k]], ssem,
                    add=True)
            for k in range(fire):
                pltpu.make_async_copy(
                    ones_v, deg_sh.at[ridx.at[ci * fire + k]], ssem).wait()
            return c

        lax.fori_loop(0, ub // fire, step, 0)
        plsc.subcore_barrier()

        @pl.when(cid == 0)
        def _():
            pltpu.sync_copy(deg_sh.at[pl.ds(sid * npt, npt)],
                            d0_hbm.at[pl.ds(sid * npt, npt)])

        @pl.when(cid == 1)
        def _():
            pltpu.sync_copy(deg_sh.at[pl.ds(sid * npt, npt)],
                            d1_hbm.at[pl.ds(sid * npt, npt)])

    return pl.kernel(
        body,
        out_type=(jax.ShapeDtypeStruct((n_pad,), F32),
                  jax.ShapeDtypeStruct((n_pad,), F32)),
        mesh=_sc_mesh(),
        scratch_types=[
            pltpu.VMEM((1, ub * BATCH), jnp.int32),
            pltpu.VMEM((ub, BATCH), jnp.int32),
            pltpu.VMEM((BATCH,), F32),
            pltpu.VMEM((npt,), F32),
            pltpu.SemaphoreType.DMA,
            pltpu.VMEM_SHARED((n_pad,), F32),
        ],
    )


def _make_spmm(n_pad, ub, cb, d):
    npt = n_pad // NS

    def body(xs_hbm, edge_hbm, a0_hbm, a1_hbm,
             eflat, cidx, ridx, rows0, rows1, s0, s1, acc_sh):
        cid = lax.axis_index("c")
        sid = lax.axis_index("s")
        wid = cid * NS + sid
        ept = ub * BATCH

        def zrow(r, c):
            for k in range(d // LANES):
                rows0[r, pl.ds(k * LANES, LANES)] = jnp.zeros((LANES,), F32)
            return c

        lax.fori_loop(0, BATCH, zrow, 0)
        for t in range(npt // BATCH):
            pltpu.sync_copy(
                rows0, acc_sh.at[pl.ds(sid * npt + t * BATCH, BATCH)])
        plsc.subcore_barrier()

        def chunk(ci, cc):
            ebase = pl.multiple_of(wid * ept + ci * cb * BATCH, BATCH)
            pltpu.sync_copy(
                edge_hbm.at[pl.ds(0, 2), pl.ds(ebase, cb * BATCH)], eflat)

            def repack(j, c):
                for i in range(BATCH // LANES):
                    sl = pl.ds(j * BATCH + i * LANES, LANES)
                    ridx[j, pl.ds(i * LANES, LANES)] = eflat[0, sl]
                    cidx[j, pl.ds(i * LANES, LANES)] = eflat[1, sl]
                return c

            lax.fori_loop(0, cb, repack, 0)
            pltpu.async_copy(xs_hbm.at[cidx.at[0]], rows0, s0)

            def step(g, c):
                j0 = 2 * g
                j1 = j0 + 1
                pltpu.async_copy(xs_hbm.at[cidx.at[j1]], rows1, s1)
                pltpu.make_async_copy(
                    xs_hbm.at[cidx.at[j0]], rows0, s0).wait()
                pltpu.sync_copy(rows0, acc_sh.at[ridx.at[j0]], add=True)

                @pl.when(g + 1 < cb // 2)
                def _():
                    pltpu.async_copy(xs_hbm.at[cidx.at[j0 + 2]], rows0, s0)

                pltpu.make_async_copy(
                    xs_hbm.at[cidx.at[j1]], rows1, s1).wait()
                pltpu.sync_copy(rows1, acc_sh.at[ridx.at[j1]], add=True)
                return c

            lax.fori_loop(0, cb // 2, step, 0)
            return cc

        lax.fori_loop(0, ub // cb, chunk, 0)
        plsc.subcore_barrier()

        @pl.when(cid == 0)
        def _():
            pltpu.sync_copy(acc_sh.at[pl.ds(sid * npt, npt)],
                            a0_hbm.at[pl.ds(sid * npt, npt)])

        @pl.when(cid == 1)
        def _():
            pltpu.sync_copy(acc_sh.at[pl.ds(sid * npt, npt)],
                            a1_hbm.at[pl.ds(sid * npt, npt)])

    return pl.kernel(
        body,
        out_type=(jax.ShapeDtypeStruct((n_pad, d), F32),
                  jax.ShapeDtypeStruct((n_pad, d), F32)),
        mesh=_sc_mesh(),
        scratch_types=[
            pltpu.VMEM((2, cb * BATCH), jnp.int32),
            pltpu.VMEM((cb, BATCH), jnp.int32),
            pltpu.VMEM((cb, BATCH), jnp.int32),
            pltpu.VMEM((BATCH, d), F32),
            pltpu.VMEM((BATCH, d), F32),
            pltpu.SemaphoreType.DMA,
            pltpu.SemaphoreType.DMA,
            pltpu.VMEM_SHARED((n_pad, d), F32),
        ],
    )


def _dis_from(d0_ref, d1_ref):
    deg = d0_ref[...] + d1_ref[...]
    return jnp.where(deg > 0, lax.rsqrt(jnp.maximum(deg, 1e-12)), 0.0)


def _scale_body(x_ref, d0_ref, d1_ref, xs_ref):
    xs_ref[...] = x_ref[...] * _dis_from(d0_ref, d1_ref)


def _linear_body(p0_ref, p1_ref, d0_ref, d1_ref, w_ref, b_ref, o_ref):
    h = (p0_ref[...] + p1_ref[...]) * _dis_from(d0_ref, d1_ref)
    o_ref[...] = lax.dot_general(
        h, w_ref[...], (((1,), (1,)), ((), ())),
        preferred_element_type=F32) + b_ref[...]


def kernel(x, edge_index, W, b):
    n, d = x.shape
    e = edge_index.shape[1]
    unit = NW * BATCH
    n_units = -(-e // unit)
    n_units += n_units % 2
    e_pad = n_units * unit
    ub = e_pad // unit

    grain = NS * BATCH
    n_pad = -(-n // grain) * grain
    if e_pad > e and n_pad == n:
        n_pad += grain

    pad = e_pad - e
    if pad:
        extra = np.arange(pad)
        epad = jnp.concatenate(
            [edge_index,
             jnp.asarray(np.stack([n + extra % (n_pad - n), extra % n]),
                         dtype=jnp.int32)], axis=1)
    else:
        epad = edge_index

    d0, d1 = _make_hist(n_pad, ub)(epad)
    d0c = d0.reshape(n_pad, 1)
    d1c = d1.reshape(n_pad, 1)

    rb = 2000 if n % 2000 == 0 else 1000
    grid = (n // rb,)
    xs = pl.pallas_call(
        _scale_body,
        grid=grid,
        in_specs=[pl.BlockSpec((rb, d), lambda i: (i, 0)),
                  pl.BlockSpec((rb, 1), lambda i: (i, 0)),
                  pl.BlockSpec((rb, 1), lambda i: (i, 0))],
        out_specs=pl.BlockSpec((rb, d), lambda i: (i, 0)),
        out_shape=jax.ShapeDtypeStruct((n, d), F32),
    )(x, d0c, d1c)

    cb = 16 if ub % 16 == 0 else 8
    a0, a1 = _make_spmm(n_pad, ub, cb, d)(xs, epad)

    out = pl.pallas_call(
        _linear_body,
        grid=grid,
        in_specs=[pl.BlockSpec((rb, d), lambda i: (i, 0)),
                  pl.BlockSpec((rb, d), lambda i: (i, 0)),
                  pl.BlockSpec((rb, 1), lambda i: (i, 0)),
                  pl.BlockSpec((rb, 1), lambda i: (i, 0)),
                  pl.BlockSpec((d, d), lambda i: (0, 0)),
                  pl.BlockSpec((1, d), lambda i: (0, 0))],
        out_specs=pl.BlockSpec((rb, d), lambda i: (i, 0)),
        out_shape=jax.ShapeDtypeStruct((n, d), F32),
    )(a0, a1, d0c, d1c, W, b.reshape(1, d))
    return out


# speedup vs baseline: 1.0265x; 1.0265x over previous
import functools

import jax
import jax.numpy as jnp
from jax import lax
from jax.experimental import pallas as pl
from jax.experimental.pallas import tpu as pltpu
from jax.experimental.pallas import tpu_sc as plsc

F32 = jnp.float32

NC = 2
NS = 16
LANES = 16
NW = NC * NS
BATCH = 128


def _sc_mesh():
    return plsc.VectorSubcoreMesh(
        core_axis_name="c", subcore_axis_name="s",
        num_cores=NC, num_subcores=NS)


def _make_hist(n_pad, nb, nx):
    npt = n_pad // NS
    fire = 16
    chunks = [(i * fire, fire) for i in range(nb // fire)]
    if nb % fire:
        chunks.append((nb - nb % fire, nb % fire))

    def body(edge_hbm, d0_hbm, d1_hbm, eflat, ridx, ones_v, zbuf, ssem,
             deg_sh):
        cid = lax.axis_index("c")
        sid = lax.axis_index("s")
        wid = cid * NS + sid
        for i in range(npt // LANES):
            zbuf[pl.ds(i * LANES, LANES)] = jnp.zeros((LANES,), F32)
        for i in range(BATCH // LANES):
            ones_v[pl.ds(i * LANES, LANES)] = jnp.ones((LANES,), F32)
        pltpu.sync_copy(zbuf, deg_sh.at[pl.ds(sid * npt, npt)])
        base_b = nb * wid + jnp.minimum(wid, nx)
        ebase = pl.multiple_of(base_b * BATCH, BATCH)
        pltpu.sync_copy(edge_hbm.at[pl.ds(0, 1), pl.ds(ebase, nb * BATCH)],
                        eflat)

        @pl.when(wid < nx)
        def _():
            xbase = pl.multiple_of((nb * NW + wid) * BATCH, BATCH)
            pltpu.sync_copy(edge_hbm.at[pl.ds(0, 1), pl.ds(xbase, BATCH)],
                            ridx.at[pl.ds(nb, 1)])

        def repack(j, c):
            for i in range(BATCH // LANES):
                ridx[j, pl.ds(i * LANES, LANES)] = (
                    eflat[0, pl.ds(j * BATCH + i * LANES, LANES)])
            return c

        lax.fori_loop(0, nb, repack, 0)
        plsc.subcore_barrier()

        def step(ci, c):
            for k in range(fire):
                pltpu.async_copy(
                    ones_v, deg_sh.at[ridx.at[ci * fire + k]], ssem,
                    add=True)
            for k in range(fire):
                pltpu.make_async_copy(
                    ones_v, deg_sh.at[ridx.at[ci * fire + k]], ssem).wait()
            return c

        lax.fori_loop(0, nb // fire, step, 0)
        off, cnt = chunks[-1] if nb % fire else (nb, 0)
        for k in range(cnt):
            pltpu.async_copy(ones_v, deg_sh.at[ridx.at[off + k]], ssem,
                             add=True)
        for k in range(cnt):
            pltpu.make_async_copy(ones_v, deg_sh.at[ridx.at[off + k]],
                                  ssem).wait()

        @pl.when(wid < nx)
        def _():
            pltpu.sync_copy(ones_v, deg_sh.at[ridx.at[nb]], add=True)

        plsc.subcore_barrier()

        @pl.when(cid == 0)
        def _():
            pltpu.sync_copy(deg_sh.at[pl.ds(sid * npt, npt)],
                            d0_hbm.at[pl.ds(sid * npt, npt)])

        @pl.when(cid == 1)
        def _():
            pltpu.sync_copy(deg_sh.at[pl.ds(sid * npt, npt)],
                            d1_hbm.at[pl.ds(sid * npt, npt)])

    return pl.kernel(
        body,
        out_type=(jax.ShapeDtypeStruct((n_pad,), F32),
                  jax.ShapeDtypeStruct((n_pad,), F32)),
        mesh=_sc_mesh(),
        scratch_types=[
            pltpu.VMEM((1, nb * BATCH), jnp.int32),
            pltpu.VMEM((nb + 1, BATCH), jnp.int32),
            pltpu.VMEM((BATCH,), F32),
            pltpu.VMEM((npt,), F32),
            pltpu.SemaphoreType.DMA,
            pltpu.VMEM_SHARED((n_pad,), F32),
        ],
    )


def _make_spmm(n_pad, nb, nx, cb, d):
    npt = n_pad // NS

    def body(xs_hbm, edge_hbm, a0_hbm, a1_hbm,
             eflat, cidx, ridx, rows0, rows1, s0, s1, acc_sh):
        cid = lax.axis_index("c")
        sid = lax.axis_index("s")
        wid = cid * NS + sid
        base_b = nb * wid + jnp.minimum(wid, nx)

        def zrow(r, c):
            for k in range(d // LANES):
                rows0[r, pl.ds(k * LANES, LANES)] = jnp.zeros((LANES,), F32)
            return c

        lax.fori_loop(0, BATCH, zrow, 0)
        for t in range(npt // BATCH):
            pltpu.sync_copy(
                rows0, acc_sh.at[pl.ds(sid * npt + t * BATCH, BATCH)])
        plsc.subcore_barrier()

        def emit_chunk(off, cnt):
            ebase = pl.multiple_of((base_b + off) * BATCH, BATCH)
            pltpu.sync_copy(
                edge_hbm.at[pl.ds(0, 2), pl.ds(ebase, cnt * BATCH)],
                eflat.at[pl.ds(0, 2), pl.ds(0, cnt * BATCH)])

            def repack(j, c):
                for i in range(BATCH // LANES):
                    sl = pl.ds(j * BATCH + i * LANES, LANES)
                    ridx[j, pl.ds(i * LANES, LANES)] = eflat[0, sl]
                    cidx[j, pl.ds(i * LANES, LANES)] = eflat[1, sl]
                return c

            lax.fori_loop(0, cnt, repack, 0)
            pltpu.async_copy(xs_hbm.at[cidx.at[0]], rows0, s0)

            def step(g, c):
                j0 = 2 * g
                j1 = j0 + 1
                pltpu.async_copy(xs_hbm.at[cidx.at[j1]], rows1, s1)
                pltpu.make_async_copy(
                    xs_hbm.at[cidx.at[j0]], rows0, s0).wait()
                pltpu.sync_copy(rows0, acc_sh.at[ridx.at[j0]], add=True)

                @pl.when(g + 1 < cnt // 2)
                def _():
                    pltpu.async_copy(xs_hbm.at[cidx.at[j0 + 2]], rows0, s0)

                pltpu.make_async_copy(
                    xs_hbm.at[cidx.at[j1]], rows1, s1).wait()
                pltpu.sync_copy(rows1, acc_sh.at[ridx.at[j1]], add=True)
                return c

            lax.fori_loop(0, cnt // 2, step, 0)

        def chunk(ci, cc):
            emit_chunk(ci * cb, cb)
            return cc

        lax.fori_loop(0, nb // cb, chunk, 0)
        if nb % cb:
            emit_chunk(nb - nb % cb, nb % cb)

        @pl.when(wid < nx)
        def _():
            xbase = pl.multiple_of((nb * NW + wid) * BATCH, BATCH)
            pltpu.sync_copy(
                edge_hbm.at[pl.ds(0, 2), pl.ds(xbase, BATCH)],
                eflat.at[pl.ds(0, 2), pl.ds(0, BATCH)])
            for i in range(BATCH // LANES):
                sl = pl.ds(i * LANES, LANES)
                ridx[0, sl] = eflat[0, sl]
                cidx[0, sl] = eflat[1, sl]
            pltpu.async_copy(xs_hbm.at[cidx.at[0]], rows0, s0)
            pltpu.make_async_copy(xs_hbm.at[cidx.at[0]], rows0, s0).wait()
            pltpu.sync_copy(rows0, acc_sh.at[ridx.at[0]], add=True)

        plsc.subcore_barrier()

        @pl.when(cid == 0)
        def _():
            pltpu.sync_copy(acc_sh.at[pl.ds(sid * npt, npt)],
                            a0_hbm.at[pl.ds(sid * npt, npt)])

        @pl.when(cid == 1)
        def _():
            pltpu.sync_copy(acc_sh.at[pl.ds(sid * npt, npt)],
                            a1_hbm.at[pl.ds(sid * npt, npt)])

    return pl.kernel(
        body,
        out_type=(jax.ShapeDtypeStruct((n_pad, d), F32),
                  jax.ShapeDtypeStruct((n_pad, d), F32)),
        mesh=_sc_mesh(),
        scratch_types=[
            pltpu.VMEM((2, cb * BATCH), jnp.int32),
            pltpu.VMEM((cb, BATCH), jnp.int32),
            pltpu.VMEM((cb, BATCH), jnp.int32),
            pltpu.VMEM((BATCH, d), F32),
            pltpu.VMEM((BATCH, d), F32),
            pltpu.SemaphoreType.DMA,
            pltpu.SemaphoreType.DMA,
            pltpu.VMEM_SHARED((n_pad, d), F32),
        ],
    )


def _dis_from(d0, d1):
    deg = d0 + d1
    return jnp.where(deg > 0, lax.rsqrt(jnp.maximum(deg, 1e-12)), 0.0)


def _scale_body(x_ref, degt_ref, xs_ref):
    xs_ref[...] = x_ref[...] * _dis_from(degt_ref[:, 0:1], degt_ref[:, 1:2])


def _linear_body(p0_ref, p1_ref, degt_ref, w_ref, b_ref, o_ref):
    h = (p0_ref[...] + p1_ref[...]) * _dis_from(
        degt_ref[:, 0:1], degt_ref[:, 1:2])
    o_ref[...] = lax.dot_general(
        h, w_ref[...], (((1,), (1,)), ((), ())),
        preferred_element_type=F32) + b_ref[...]


def kernel(x, edge_index, W, b):
    n, d = x.shape
    e = edge_index.shape[1]
    assert e % BATCH == 0
    nbt = e // BATCH
    nb = nbt // NW
    nx = nbt % NW
    assert nb % 2 == 0

    grain = NS * BATCH
    n_pad = -(-n // grain) * grain

    d0, d1 = _make_hist(n_pad, nb, nx)(edge_index)
    degt = jnp.stack([d0[:n], d1[:n]], axis=1)

    rb = 2000 if n % 2000 == 0 else 1000
    grid = (n // rb,)
    xs = pl.pallas_call(
        _scale_body,
        grid=grid,
        in_specs=[pl.BlockSpec((rb, d), lambda i: (i, 0)),
                  pl.BlockSpec((rb, 2), lambda i: (i, 0))],
        out_specs=pl.BlockSpec((rb, d), lambda i: (i, 0)),
        out_shape=jax.ShapeDtypeStruct((n, d), F32),
    )(x, degt)

    a0, a1 = _make_spmm(n_pad, nb, nx, 16, d)(xs, edge_index)

    out = pl.pallas_call(
        _linear_body,
        grid=grid,
        in_specs=[pl.BlockSpec((rb, d), lambda i: (i, 0)),
                  pl.BlockSpec((rb, d), lambda i: (i, 0)),
                  pl.BlockSpec((rb, 2), lambda i: (i, 0)),
                  pl.BlockSpec((d, d), lambda i: (0, 0)),
                  pl.BlockSpec((1, d), lambda i: (0, 0))],
        out_specs=pl.BlockSpec((rb, d), lambda i: (i, 0)),
        out_shape=jax.ShapeDtypeStruct((n, d), F32),
    )(a0, a1, degt, W, b.reshape(1, d))
    return out
